# baseline (speedup 1.0000x reference)
import functools
import math

import jax
import jax.numpy as jnp
from jax import lax
from jax.experimental import pallas as pl
from jax.experimental.pallas import tpu as pltpu
from jax.experimental.pallas import tpu_sc as plsc

N = 10000
E = 320000
H = 128
NR = 32
FD = 64
CUT = 5.0
LN2 = math.log(2.0)
OFF_STEP = CUT / (NR - 1)
COEFF = -0.5 / OFF_STEP**2

NC, NS = 2, 16
NW = NC * NS
EPW = E // NW
CHUNK = 80
CPW = EPW // CHUNK
RPS = 624
TAIL = N - RPS * NS
TOFF = RPS * NS

BE = 2000
BN = 2000


def _sp(v):
    return jnp.maximum(v, 0.0) + jnp.log1p(jnp.exp(-jnp.abs(v))) - LN2


def _sig(v):
    return 1.0 / (1.0 + jnp.exp(-v))


def _dist_env(ew):
    d2 = jnp.sum(ew * ew, axis=1, keepdims=True)
    d = jnp.sqrt(d2)
    offs = lax.broadcasted_iota(jnp.int32, (1, NR), 1).astype(jnp.float32) * OFF_STEP
    ea = jnp.exp(COEFF * (d - offs) ** 2)
    u = jnp.clip(d / CUT, 0.0, 1.0)
    u2 = u * u
    u3 = u2 * u
    env = 1.0 - (6.0 * u3 * u2 - 15.0 * u2 * u2 + 10.0 * u3)
    return d, offs, ea, u, env


def _filter_fwd_body(ew_ref, w1_ref, b1_ref, w2p_ref, b2p_ref, wf0_ref, wf1_ref):
    ew = ew_ref[...]
    _, _, ea, _, env = _dist_env(ew)
    outs = (wf0_ref, wf1_ref)
    for t in range(2):
        a = jnp.dot(ea, w1_ref[t], preferred_element_type=jnp.float32, precision=lax.Precision.HIGHEST) + b1_ref[t : t + 1, :]
        f = jnp.dot(_sp(a), w2p_ref[t], preferred_element_type=jnp.float32, precision=lax.Precision.HIGHEST) + b2p_ref[t : t + 1, :]
        outs[t][...] = f * env


def _filter_fwd(ew, w1, b1, w2p, b2p):
    return pl.pallas_call(
        _filter_fwd_body,
        grid=(E // BE,),
        in_specs=[
            pl.BlockSpec((BE, 3), lambda i: (i, 0)),
            pl.BlockSpec((2, NR, FD), lambda i: (0, 0, 0)),
            pl.BlockSpec((2, FD), lambda i: (0, 0)),
            pl.BlockSpec((2, FD, H), lambda i: (0, 0, 0)),
            pl.BlockSpec((2, H), lambda i: (0, 0)),
        ],
        out_specs=[
            pl.BlockSpec((BE, H), lambda i: (i, 0)),
            pl.BlockSpec((BE, H), lambda i: (i, 0)),
        ],
        out_shape=[
            jax.ShapeDtypeStruct((E, H), jnp.float32),
            jax.ShapeDtypeStruct((E, H), jnp.float32),
        ],
    )(ew, w1, b1, w2p, b2p)


def _emb_body(x_ref, emb_ref, l1wp_ref, h0_ref, hp0_ref):
    xi = x_ref[...]
    ids = lax.broadcasted_iota(jnp.int32, (BN, H), 1)
    oh = (xi == ids).astype(jnp.float32)
    h0 = jnp.dot(oh, emb_ref[...], preferred_element_type=jnp.float32, precision=lax.Precision.HIGHEST)
    h0_ref[...] = h0
    hp0_ref[...] = jnp.dot(h0, l1wp_ref[...], preferred_element_type=jnp.float32, precision=lax.Precision.HIGHEST)


def _emb(x_col, emb_pad, l1w0p):
    return pl.pallas_call(
        _emb_body,
        grid=(N // BN,),
        in_specs=[
            pl.BlockSpec((BN, 1), lambda i: (i, 0)),
            pl.BlockSpec((H, H), lambda i: (0, 0)),
            pl.BlockSpec((H, H), lambda i: (0, 0)),
        ],
        out_specs=[
            pl.BlockSpec((BN, H), lambda i: (i, 0)),
            pl.BlockSpec((BN, H), lambda i: (i, 0)),
        ],
        out_shape=[
            jax.ShapeDtypeStruct((N, H), jnp.float32),
            jax.ShapeDtypeStruct((N, H), jnp.float32),
        ],
    )(x_col, emb_pad, l1w0p)


def _node1_body(h0_ref, ap_ref, l2wp_ref, l2b_ref, l1wp_ref, h1_ref, hp1_ref, agg0_ref):
    agg = ap_ref[0] + ap_ref[1]
    z2 = jnp.dot(agg, l2wp_ref[...], preferred_element_type=jnp.float32, precision=lax.Precision.HIGHEST) + l2b_ref[...]
    h1 = h0_ref[...] + _sp(z2)
    h1_ref[...] = h1
    hp1_ref[...] = jnp.dot(h1, l1wp_ref[...], preferred_element_type=jnp.float32, precision=lax.Precision.HIGHEST)
    agg0_ref[...] = agg


def _node1(h0, aggp, l2w0p, l2b0, l1w1p):
    return pl.pallas_call(
        _node1_body,
        grid=(N // BN,),
        in_specs=[
            pl.BlockSpec((BN, H), lambda i: (i, 0)),
            pl.BlockSpec((2, BN, H), lambda i: (0, i, 0)),
            pl.BlockSpec((H, H), lambda i: (0, 0)),
            pl.BlockSpec((1, H), lambda i: (0, 0)),
            pl.BlockSpec((H, H), lambda i: (0, 0)),
        ],
        out_specs=[
            pl.BlockSpec((BN, H), lambda i: (i, 0)),
            pl.BlockSpec((BN, H), lambda i: (i, 0)),
            pl.BlockSpec((BN, H), lambda i: (i, 0)),
        ],
        out_shape=[
            jax.ShapeDtypeStruct((N, H), jnp.float32),
            jax.ShapeDtypeStruct((N, H), jnp.float32),
            jax.ShapeDtypeStruct((N, H), jnp.float32),
        ],
    )(h0, aggp, l2w0p, l2b0, l1w1p)


def _node2_body(h1_ref, ap_ref, l2wp_ref, l2b_ref, ow1_ref, ob1_ref, ow2r_ref,
                ow1t_ref, l2wtp_ref, gh2_ref, gagg1_ref, es_ref):
    i = pl.program_id(0)
    agg = ap_ref[0] + ap_ref[1]
    z2 = jnp.dot(agg, l2wp_ref[...], preferred_element_type=jnp.float32, precision=lax.Precision.HIGHEST) + l2b_ref[...]
    h2 = h1_ref[...] + _sp(z2)
    z1 = jnp.dot(h2, ow1_ref[...], preferred_element_type=jnp.float32, precision=lax.Precision.HIGHEST) + ob1_ref[...]
    w2row = ow2r_ref[...]
    e_node = jnp.sum(_sp(z1) * w2row, axis=1, keepdims=True)
    s = jnp.sum(e_node)
    r0 = lax.broadcasted_iota(jnp.int32, (8, 128), 0) == 0
    c0 = lax.broadcasted_iota(jnp.int32, (8, 128), 1) == 0
    contrib = jnp.where(r0 & c0, s, 0.0)

    @pl.when(i == 0)
    def _():
        es_ref[...] = contrib

    @pl.when(i > 0)
    def _():
        es_ref[...] = es_ref[...] + contrib

    gz1 = _sig(z1) * w2row
    gh2 = jnp.dot(gz1, ow1t_ref[...], preferred_element_type=jnp.float32, precision=lax.Precision.HIGHEST)
    gh2_ref[...] = gh2
    gagg1_ref[...] = jnp.dot(gh2 * _sig(z2), l2wtp_ref[...], preferred_element_type=jnp.float32, precision=lax.Precision.HIGHEST)


def _node2(h1, aggp, l2w1p, l2b1, ow1, ob1, ow2r, ow1t, l2w1tp):
    return pl.pallas_call(
        _node2_body,
        grid=(N // BN,),
        in_specs=[
            pl.BlockSpec((BN, H), lambda i: (i, 0)),
            pl.BlockSpec((2, BN, H), lambda i: (0, i, 0)),
            pl.BlockSpec((H, H), lambda i: (0, 0)),
            pl.BlockSpec((1, H), lambda i: (0, 0)),
            pl.BlockSpec((H, FD), lambda i: (0, 0)),
            pl.BlockSpec((1, FD), lambda i: (0, 0)),
            pl.BlockSpec((1, FD), lambda i: (0, 0)),
            pl.BlockSpec((FD, H), lambda i: (0, 0)),
            pl.BlockSpec((H, H), lambda i: (0, 0)),
        ],
        out_specs=[
            pl.BlockSpec((BN, H), lambda i: (i, 0)),
            pl.BlockSpec((BN, H), lambda i: (i, 0)),
            pl.BlockSpec((8, 128), lambda i: (0, 0)),
        ],
        out_shape=[
            jax.ShapeDtypeStruct((N, H), jnp.float32),
            jax.ShapeDtypeStruct((N, H), jnp.float32),
            jax.ShapeDtypeStruct((8, 128), jnp.float32),
        ],
        compiler_params=pltpu.CompilerParams(dimension_semantics=("arbitrary",)),
    )(h1, aggp, l2w1p, l2b1, ow1, ob1, ow2r, ow1t, l2w1tp)


def _node3_body(gh2_ref, gp_ref, l1wtp_ref, agg0_ref, l2wp_ref, l2b_ref, l2wtp_ref, gagg0_ref):
    gproj = gp_ref[0] + gp_ref[1]
    gh1 = gh2_ref[...] + jnp.dot(gproj, l1wtp_ref[...], preferred_element_type=jnp.float32, precision=lax.Precision.HIGHEST)
    z2 = jnp.dot(agg0_ref[...], l2wp_ref[...], preferred_element_type=jnp.float32, precision=lax.Precision.HIGHEST) + l2b_ref[...]
    gagg0_ref[...] = jnp.dot(gh1 * _sig(z2), l2wtp_ref[...], preferred_element_type=jnp.float32, precision=lax.Precision.HIGHEST)


def _node3(gh2, gpp, l1w1tp, agg0, l2w0p, l2b0, l2w0tp):
    return pl.pallas_call(
        _node3_body,
        grid=(N // BN,),
        in_specs=[
            pl.BlockSpec((BN, H), lambda i: (i, 0)),
            pl.BlockSpec((2, BN, H), lambda i: (0, i, 0)),
            pl.BlockSpec((H, H), lambda i: (0, 0)),
            pl.BlockSpec((BN, H), lambda i: (i, 0)),
            pl.BlockSpec((H, H), lambda i: (0, 0)),
            pl.BlockSpec((1, H), lambda i: (0, 0)),
            pl.BlockSpec((H, H), lambda i: (0, 0)),
        ],
        out_specs=pl.BlockSpec((BN, H), lambda i: (i, 0)),
        out_shape=jax.ShapeDtypeStruct((N, H), jnp.float32),
    )(gh2, gpp, l1w1tp, agg0, l2w0p, l2b0, l2w0tp)


def _filter_bwd_body(ew_ref, gw0_ref, gw1_ref, w1_ref, b1_ref, w2p_ref, b2p_ref,
                     w1t_ref, w2tp_ref, de_ref, sig_ref):
    i = pl.program_id(0)
    ew = ew_ref[...]
    d, offs, ea, u, env = _dist_env(ew)
    gws = (gw0_ref, gw1_ref)
    g_ea = jnp.zeros((BE, NR), jnp.float32)
    g_env = jnp.zeros((BE, 1), jnp.float32)
    for t in range(2):
        a = jnp.dot(ea, w1_ref[t], preferred_element_type=jnp.float32, precision=lax.Precision.HIGHEST) + b1_ref[t : t + 1, :]
        f = jnp.dot(_sp(a), w2p_ref[t], preferred_element_type=jnp.float32, precision=lax.Precision.HIGHEST) + b2p_ref[t : t + 1, :]
        gw = gws[t][...]
        g_env = g_env + jnp.sum(gw * f, axis=1, keepdims=True)
        ga = jnp.dot(gw * env, w2tp_ref[t], preferred_element_type=jnp.float32, precision=lax.Precision.HIGHEST) * _sig(a)
        g_ea = g_ea + jnp.dot(ga, w1t_ref[t], preferred_element_type=jnp.float32, precision=lax.Precision.HIGHEST)
    dea = 2.0 * COEFF * (d - offs) * ea
    u2 = u * u
    denv = jnp.where(d <= CUT, -(30.0 * u2 * u2 - 60.0 * u2 * u + 30.0 * u2) / CUT, 0.0)
    g_d = jnp.sum(g_ea * dea, axis=1, keepdims=True) + g_env * denv
    gew = (g_d / d) * ew
    de = jnp.concatenate([gew, jnp.zeros((BE, H - 3), jnp.float32)], axis=1)
    de_ref[...] = de
    r0 = lax.broadcasted_iota(jnp.int32, (8, 128), 0) == 0
    lanes = lax.broadcasted_iota(jnp.int32, (8, 128), 1)
    total = jnp.zeros((8, 128), jnp.float32)
    for k in range(3):
        for l in range(3):
            skl = jnp.sum(ew[:, k : k + 1] * gew[:, l : l + 1])
            total = total + jnp.where(r0 & (lanes == 3 * k + l), skl, 0.0)

    @pl.when(i == 0)
    def _():
        sig_ref[...] = total

    @pl.when(i > 0)
    def _():
        sig_ref[...] = sig_ref[...] + total


def _filter_bwd(ew, gw0, gw1, w1, b1, w2p, b2p, w1t, w2tp):
    return pl.pallas_call(
        _filter_bwd_body,
        grid=(E // BE,),
        in_specs=[
            pl.BlockSpec((BE, 3), lambda i: (i, 0)),
            pl.BlockSpec((BE, H), lambda i: (i, 0)),
            pl.BlockSpec((BE, H), lambda i: (i, 0)),
            pl.BlockSpec((2, NR, FD), lambda i: (0, 0, 0)),
            pl.BlockSpec((2, FD), lambda i: (0, 0)),
            pl.BlockSpec((2, FD, H), lambda i: (0, 0, 0)),
            pl.BlockSpec((2, H), lambda i: (0, 0)),
            pl.BlockSpec((2, FD, NR), lambda i: (0, 0, 0)),
            pl.BlockSpec((2, H, FD), lambda i: (0, 0, 0)),
        ],
        out_specs=[
            pl.BlockSpec((BE, H), lambda i: (i, 0)),
            pl.BlockSpec((8, 128), lambda i: (0, 0)),
        ],
        out_shape=[
            jax.ShapeDtypeStruct((E, H), jnp.float32),
            jax.ShapeDtypeStruct((8, 128), jnp.float32),
        ],
        compiler_params=pltpu.CompilerParams(dimension_semantics=("arbitrary",)),
    )(ew, gw0, gw1, w1, b1, w2p, b2p, w1t, w2tp)


_sc_mesh = plsc.VectorSubcoreMesh(core_axis_name="c", subcore_axis_name="s")


@functools.partial(
    pl.kernel,
    out_type=jax.ShapeDtypeStruct((NC, N, H), jnp.float32),
    mesh=_sc_mesh,
    scratch_types=[
        pltpu.VMEM((CHUNK,), jnp.int32),
        pltpu.VMEM((CHUNK,), jnp.int32),
        pltpu.VMEM((CHUNK, H), jnp.float32),
        pltpu.VMEM((CHUNK, H), jnp.float32),
        pltpu.VMEM_SHARED((N, H), jnp.float32),
        pltpu.SemaphoreType.DMA,
    ],
)
def _sc_fwd(hproj, wf, ii1, jj1, zn, out, ii_c, jj_c, rows_c, wf_c, agg_sh, sem):
    cid = lax.axis_index("c")
    sid = lax.axis_index("s")
    wid = sid * NC + cid
    r0 = sid * RPS
    pltpu.sync_copy(zn.at[pl.ds(r0, RPS)], agg_sh.at[pl.ds(r0, RPS)])

    @pl.when(sid == NS - 1)
    def _():
        pltpu.sync_copy(zn.at[pl.ds(TOFF, TAIL)], agg_sh.at[pl.ds(TOFF, TAIL)])

    plsc.subcore_barrier()

    def blk(b, carry):
        cbase = wid * EPW + b * CHUNK
        pltpu.sync_copy(ii1.at[pl.ds(cbase, CHUNK)], ii_c)
        pltpu.sync_copy(jj1.at[pl.ds(cbase, CHUNK)], jj_c)
        pltpu.sync_copy(wf.at[pl.ds(cbase, CHUNK)], wf_c)
        pltpu.async_copy(hproj.at[jj_c], rows_c, sem).wait()

        def mul(e, c2):
            for c in range(FD // 16):
                s = pl.ds(c * 16, 16)
                wf_c[e, s] = rows_c[e, s] * wf_c[e, s]
            return c2

        lax.fori_loop(0, CHUNK, mul, 0)
        pltpu.sync_copy(wf_c, agg_sh.at[ii_c], add=True)
        return carry

    lax.fori_loop(0, CPW, blk, 0)
    plsc.subcore_barrier()
    pltpu.sync_copy(agg_sh.at[pl.ds(r0, RPS)], out.at[cid, pl.ds(r0, RPS)])

    @pl.when(sid == NS - 1)
    def _():
        pltpu.sync_copy(agg_sh.at[pl.ds(TOFF, TAIL)], out.at[cid, pl.ds(TOFF, TAIL)])


@functools.partial(
    pl.kernel,
    out_type=[
        jax.ShapeDtypeStruct((E, H), jnp.float32),
        jax.ShapeDtypeStruct((NC, N, H), jnp.float32),
    ],
    mesh=_sc_mesh,
    scratch_types=[
        pltpu.VMEM((CHUNK,), jnp.int32),
        pltpu.VMEM((CHUNK,), jnp.int32),
        pltpu.VMEM((CHUNK, H), jnp.float32),
        pltpu.VMEM((CHUNK, H), jnp.float32),
        pltpu.VMEM((CHUNK, H), jnp.float32),
        pltpu.VMEM_SHARED((N, H), jnp.float32),
        pltpu.SemaphoreType.DMA,
    ],
)
def _sc_bwd1(gagg, hproj, wf, ii1, jj1, zn, gw_out, gp_out,
             ii_c, jj_c, ga_c, hp_c, wf_c, gp_sh, sem):
    cid = lax.axis_index("c")
    sid = lax.axis_index("s")
    wid = sid * NC + cid
    r0 = sid * RPS
    pltpu.sync_copy(zn.at[pl.ds(r0, RPS)], gp_sh.at[pl.ds(r0, RPS)])

    @pl.when(sid == NS - 1)
    def _():
        pltpu.sync_copy(zn.at[pl.ds(TOFF, TAIL)], gp_sh.at[pl.ds(TOFF, TAIL)])

    plsc.subcore_barrier()

    def blk(b, carry):
        cbase = wid * EPW + b * CHUNK
        pltpu.sync_copy(ii1.at[pl.ds(cbase, CHUNK)], ii_c)
        pltpu.sync_copy(jj1.at[pl.ds(cbase, CHUNK)], jj_c)
        pltpu.sync_copy(wf.at[pl.ds(cbase, CHUNK)], wf_c)
        cp1 = pltpu.async_copy(gagg.at[ii_c], ga_c, sem)
        cp2 = pltpu.async_copy(hproj.at[jj_c], hp_c, sem)
        cp1.wait()
        cp2.wait()

        def mul(e, c2):
            for c in range(FD // 16):
                s = pl.ds(c * 16, 16)
                ga = ga_c[e, s]
                hp_c[e, s] = ga * hp_c[e, s]
                wf_c[e, s] = ga * wf_c[e, s]
            return c2

        lax.fori_loop(0, CHUNK, mul, 0)
        pltpu.sync_copy(hp_c, gw_out.at[pl.ds(cbase, CHUNK)])
        pltpu.sync_copy(wf_c, gp_sh.at[jj_c], add=True)
        return carry

    lax.fori_loop(0, CPW, blk, 0)
    plsc.subcore_barrier()
    pltpu.sync_copy(gp_sh.at[pl.ds(r0, RPS)], gp_out.at[cid, pl.ds(r0, RPS)])

    @pl.when(sid == NS - 1)
    def _():
        pltpu.sync_copy(gp_sh.at[pl.ds(TOFF, TAIL)], gp_out.at[cid, pl.ds(TOFF, TAIL)])


@functools.partial(
    pl.kernel,
    out_type=jax.ShapeDtypeStruct((E, H), jnp.float32),
    mesh=_sc_mesh,
    scratch_types=[
        pltpu.VMEM((CHUNK,), jnp.int32),
        pltpu.VMEM((CHUNK,), jnp.int32),
        pltpu.VMEM((CHUNK, H), jnp.float32),
        pltpu.VMEM((CHUNK, H), jnp.float32),
        pltpu.SemaphoreType.DMA,
    ],
)
def _sc_bwd0(gagg, hproj, ii1, jj1, gw_out, ii_c, jj_c, ga_c, hp_c, sem):
    cid = lax.axis_index("c")
    sid = lax.axis_index("s")
    wid = sid * NC + cid

    def blk(b, carry):
        cbase = wid * EPW + b * CHUNK
        pltpu.sync_copy(ii1.at[pl.ds(cbase, CHUNK)], ii_c)
        pltpu.sync_copy(jj1.at[pl.ds(cbase, CHUNK)], jj_c)
        cp1 = pltpu.async_copy(gagg.at[ii_c], ga_c, sem)
        cp2 = pltpu.async_copy(hproj.at[jj_c], hp_c, sem)
        cp1.wait()
        cp2.wait()

        def mul(e, c2):
            for c in range(FD // 16):
                s = pl.ds(c * 16, 16)
                hp_c[e, s] = ga_c[e, s] * hp_c[e, s]
            return c2

        lax.fori_loop(0, CHUNK, mul, 0)
        pltpu.sync_copy(hp_c, gw_out.at[pl.ds(cbase, CHUNK)])
        return carry

    lax.fori_loop(0, CPW, blk, 0)


@functools.partial(
    pl.kernel,
    out_type=jax.ShapeDtypeStruct((NC, N, H), jnp.float32),
    mesh=_sc_mesh,
    scratch_types=[
        pltpu.VMEM((CHUNK,), jnp.int32),
        pltpu.VMEM((CHUNK,), jnp.int32),
        pltpu.VMEM((CHUNK, H), jnp.float32),
        pltpu.VMEM((CHUNK, H), jnp.float32),
        pltpu.VMEM_SHARED((N, H), jnp.float32),
        pltpu.SemaphoreType.DMA,
    ],
)
def _sc_force(de, ii1, jj1, zn, out, ii_c, jj_c, de_c, nde_c, f_sh, sem):
    cid = lax.axis_index("c")
    sid = lax.axis_index("s")
    wid = sid * NC + cid
    r0 = sid * RPS
    pltpu.sync_copy(zn.at[pl.ds(r0, RPS)], f_sh.at[pl.ds(r0, RPS)])

    @pl.when(sid == NS - 1)
    def _():
        pltpu.sync_copy(zn.at[pl.ds(TOFF, TAIL)], f_sh.at[pl.ds(TOFF, TAIL)])

    def z0(e, c2):
        for c in range(H // 16):
            nde_c[e, pl.ds(c * 16, 16)] = jnp.zeros((16,), jnp.float32)
        return c2

    lax.fori_loop(0, CHUNK, z0, 0)
    plsc.subcore_barrier()

    def blk(b, carry):
        cbase = wid * EPW + b * CHUNK
        pltpu.sync_copy(ii1.at[pl.ds(cbase, CHUNK)], ii_c)
        pltpu.sync_copy(jj1.at[pl.ds(cbase, CHUNK)], jj_c)
        pltpu.sync_copy(de.at[pl.ds(cbase, CHUNK)], de_c)

        def neg(e, c2):
            nde_c[e, pl.ds(0, 16)] = -de_c[e, pl.ds(0, 16)]
            return c2

        lax.fori_loop(0, CHUNK, neg, 0)
        pltpu.sync_copy(de_c, f_sh.at[ii_c], add=True)
        pltpu.sync_copy(nde_c, f_sh.at[jj_c], add=True)
        return carry

    lax.fori_loop(0, CPW, blk, 0)
    plsc.subcore_barrier()
    pltpu.sync_copy(f_sh.at[pl.ds(r0, RPS)], out.at[cid, pl.ds(r0, RPS)])

    @pl.when(sid == NS - 1)
    def _():
        pltpu.sync_copy(f_sh.at[pl.ds(TOFF, TAIL)], out.at[cid, pl.ds(TOFF, TAIL)])


def _padc(w, cols):
    return jnp.pad(w, ((0, 0), (0, cols - w.shape[1])))


def _padr(w, rows):
    return jnp.pad(w, ((0, rows - w.shape[0]), (0, 0)))


def kernel(x, edge_index, edge_weight, emb, mlp_W1, mlp_b1, mlp_W2, mlp_b2,
           lin1_W, lin2_W, lin2_b, out_W1, out_b1, out_W2, out_b2):
    ii1 = edge_index[0].astype(jnp.int32)
    jj1 = edge_index[1].astype(jnp.int32)
    x_col = x.astype(jnp.int32).reshape(N, 1)
    emb_pad = jnp.zeros((H, H), jnp.float32).at[: emb.shape[0]].set(emb)
    zn = jnp.zeros((N, H), jnp.float32)

    w2p = jnp.pad(mlp_W2, ((0, 0), (0, 0), (0, H - FD)))
    b2p = jnp.pad(mlp_b2, ((0, 0), (0, H - FD)))
    w1t = jnp.swapaxes(mlp_W1, 1, 2)
    w2tp = jnp.pad(jnp.swapaxes(mlp_W2, 1, 2), ((0, 0), (0, H - FD), (0, 0)))
    l1w0p = _padc(lin1_W[0], H)
    l1w1p = _padc(lin1_W[1], H)
    l2w0p = _padr(lin2_W[0], H)
    l2w1p = _padr(lin2_W[1], H)
    l2w0tp = _padc(lin2_W[0].T, H)
    l2w1tp = _padc(lin2_W[1].T, H)
    l1w1tp = _padr(lin1_W[1].T, H)

    wf0, wf1 = _filter_fwd(edge_weight, mlp_W1, mlp_b1, w2p, b2p)
    h0, hp0 = _emb(x_col, emb_pad, l1w0p)
    aggp0 = _sc_fwd(hp0, wf0, ii1, jj1, zn)
    h1, hp1, agg0 = _node1(h0, aggp0, l2w0p, lin2_b[0:1], l1w1p)
    aggp1 = _sc_fwd(hp1, wf1, ii1, jj1, zn)
    gh2, gagg1, es_blk = _node2(
        h1, aggp1, l2w1p, lin2_b[1:2], out_W1, out_b1.reshape(1, FD),
        out_W2.T, out_W1.T, l2w1tp)
    gw1, gpp = _sc_bwd1(gagg1, hp1, wf1, ii1, jj1, zn)
    gagg0 = _node3(gh2, gpp, l1w1tp, agg0, l2w0p, lin2_b[0:1], l2w0tp)
    gw0 = _sc_bwd0(gagg0, hp0, ii1, jj1)
    de, sig_blk = _filter_bwd(edge_weight, gw0, gw1, mlp_W1, mlp_b1,
                              w2p, b2p, w1t, w2tp)
    fpart = _sc_force(de, ii1, jj1, zn)

    e_sum = es_blk[0, 0] + N * out_b2[0]
    forces = (fpart[0] + fpart[1])[:, :3]
    sig = sig_blk[0, :9].reshape(3, 3)
    sigma = 0.5 * (sig + sig.T)
    return (e_sum, forces, sigma)


# speedup vs baseline: 1.4404x; 1.4404x over previous
import functools
import math

import jax
import jax.numpy as jnp
from jax import lax
from jax.experimental import pallas as pl
from jax.experimental.pallas import tpu as pltpu
from jax.experimental.pallas import tpu_sc as plsc

N = 10000
E = 320000
H = 128
NR = 32
FD = 64
CUT = 5.0
LN2 = math.log(2.0)
OFF_STEP = CUT / (NR - 1)
COEFF = -0.5 / OFF_STEP**2

NC, NS = 2, 16
NW = NC * NS
EPW = E // NW
CHUNK = 80
CPW = EPW // CHUNK
RPS = 624
TAIL = N - RPS * NS
TOFF = RPS * NS

BE = 2000
BN = 2000


def _sp(v):
    return jnp.maximum(v, 0.0) + jnp.log1p(jnp.exp(-jnp.abs(v))) - LN2


def _sig(v):
    return 1.0 / (1.0 + jnp.exp(-v))


def _dist_env(ew):
    d2 = jnp.sum(ew * ew, axis=1, keepdims=True)
    d = jnp.sqrt(d2)
    offs = lax.broadcasted_iota(jnp.int32, (1, NR), 1).astype(jnp.float32) * OFF_STEP
    ea = jnp.exp(COEFF * (d - offs) ** 2)
    u = jnp.clip(d / CUT, 0.0, 1.0)
    u2 = u * u
    u3 = u2 * u
    env = 1.0 - (6.0 * u3 * u2 - 15.0 * u2 * u2 + 10.0 * u3)
    return d, offs, ea, u, env


def _filter_fwd_body(ew_ref, w1_ref, b1_ref, w2p_ref, b2p_ref, wf0_ref, wf1_ref):
    ew = ew_ref[...]
    _, _, ea, _, env = _dist_env(ew)
    outs = (wf0_ref, wf1_ref)
    for t in range(2):
        a = jnp.dot(ea, w1_ref[t], preferred_element_type=jnp.float32) + b1_ref[t : t + 1, :]
        f = jnp.dot(_sp(a), w2p_ref[t], preferred_element_type=jnp.float32) + b2p_ref[t : t + 1, :]
        outs[t][...] = f * env


def _filter_fwd(ew, w1, b1, w2p, b2p):
    return pl.pallas_call(
        _filter_fwd_body,
        grid=(E // BE,),
        in_specs=[
            pl.BlockSpec((BE, 3), lambda i: (i, 0)),
            pl.BlockSpec((2, NR, FD), lambda i: (0, 0, 0)),
            pl.BlockSpec((2, FD), lambda i: (0, 0)),
            pl.BlockSpec((2, FD, H), lambda i: (0, 0, 0)),
            pl.BlockSpec((2, H), lambda i: (0, 0)),
        ],
        out_specs=[
            pl.BlockSpec((BE, H), lambda i: (i, 0)),
            pl.BlockSpec((BE, H), lambda i: (i, 0)),
        ],
        out_shape=[
            jax.ShapeDtypeStruct((E, H), jnp.float32),
            jax.ShapeDtypeStruct((E, H), jnp.float32),
        ],
    )(ew, w1, b1, w2p, b2p)


def _emb_body(x_ref, emb_ref, l1wp_ref, h0_ref, hp0_ref):
    xi = x_ref[...]
    ids = lax.broadcasted_iota(jnp.int32, (BN, H), 1)
    oh = (xi == ids).astype(jnp.float32)
    h0 = jnp.dot(oh, emb_ref[...], preferred_element_type=jnp.float32, precision=lax.Precision.HIGHEST)
    h0_ref[...] = h0
    hp0_ref[...] = jnp.dot(h0, l1wp_ref[...], preferred_element_type=jnp.float32, precision=lax.Precision.HIGHEST)


def _emb(x_col, emb_pad, l1w0p):
    return pl.pallas_call(
        _emb_body,
        grid=(N // BN,),
        in_specs=[
            pl.BlockSpec((BN, 1), lambda i: (i, 0)),
            pl.BlockSpec((H, H), lambda i: (0, 0)),
            pl.BlockSpec((H, H), lambda i: (0, 0)),
        ],
        out_specs=[
            pl.BlockSpec((BN, H), lambda i: (i, 0)),
            pl.BlockSpec((BN, H), lambda i: (i, 0)),
        ],
        out_shape=[
            jax.ShapeDtypeStruct((N, H), jnp.float32),
            jax.ShapeDtypeStruct((N, H), jnp.float32),
        ],
    )(x_col, emb_pad, l1w0p)


def _node1_body(h0_ref, ap_ref, l2wp_ref, l2b_ref, l1wp_ref, h1_ref, hp1_ref, agg0_ref):
    agg = ap_ref[0] + ap_ref[1]
    z2 = jnp.dot(agg, l2wp_ref[...], preferred_element_type=jnp.float32, precision=lax.Precision.HIGHEST) + l2b_ref[...]
    h1 = h0_ref[...] + _sp(z2)
    h1_ref[...] = h1
    hp1_ref[...] = jnp.dot(h1, l1wp_ref[...], preferred_element_type=jnp.float32, precision=lax.Precision.HIGHEST)
    agg0_ref[...] = agg


def _node1(h0, aggp, l2w0p, l2b0, l1w1p):
    return pl.pallas_call(
        _node1_body,
        grid=(N // BN,),
        in_specs=[
            pl.BlockSpec((BN, H), lambda i: (i, 0)),
            pl.BlockSpec((2, BN, H), lambda i: (0, i, 0)),
            pl.BlockSpec((H, H), lambda i: (0, 0)),
            pl.BlockSpec((1, H), lambda i: (0, 0)),
            pl.BlockSpec((H, H), lambda i: (0, 0)),
        ],
        out_specs=[
            pl.BlockSpec((BN, H), lambda i: (i, 0)),
            pl.BlockSpec((BN, H), lambda i: (i, 0)),
            pl.BlockSpec((BN, H), lambda i: (i, 0)),
        ],
        out_shape=[
            jax.ShapeDtypeStruct((N, H), jnp.float32),
            jax.ShapeDtypeStruct((N, H), jnp.float32),
            jax.ShapeDtypeStruct((N, H), jnp.float32),
        ],
    )(h0, aggp, l2w0p, l2b0, l1w1p)


def _node2_body(h1_ref, ap_ref, l2wp_ref, l2b_ref, ow1_ref, ob1_ref, ow2r_ref,
                ow1t_ref, l2wtp_ref, gh2_ref, gagg1_ref, es_ref):
    i = pl.program_id(0)
    agg = ap_ref[0] + ap_ref[1]
    z2 = jnp.dot(agg, l2wp_ref[...], preferred_element_type=jnp.float32, precision=lax.Precision.HIGHEST) + l2b_ref[...]
    h2 = h1_ref[...] + _sp(z2)
    z1 = jnp.dot(h2, ow1_ref[...], preferred_element_type=jnp.float32, precision=lax.Precision.HIGHEST) + ob1_ref[...]
    w2row = ow2r_ref[...]
    e_node = jnp.sum(_sp(z1) * w2row, axis=1, keepdims=True)
    s = jnp.sum(e_node)
    r0 = lax.broadcasted_iota(jnp.int32, (8, 128), 0) == 0
    c0 = lax.broadcasted_iota(jnp.int32, (8, 128), 1) == 0
    contrib = jnp.where(r0 & c0, s, 0.0)

    @pl.when(i == 0)
    def _():
        es_ref[...] = contrib

    @pl.when(i > 0)
    def _():
        es_ref[...] = es_ref[...] + contrib

    gz1 = _sig(z1) * w2row
    gh2 = jnp.dot(gz1, ow1t_ref[...], preferred_element_type=jnp.float32, precision=lax.Precision.HIGHEST)
    gh2_ref[...] = gh2
    gagg1_ref[...] = jnp.dot(gh2 * _sig(z2), l2wtp_ref[...], preferred_element_type=jnp.float32, precision=lax.Precision.HIGHEST)


def _node2(h1, aggp, l2w1p, l2b1, ow1, ob1, ow2r, ow1t, l2w1tp):
    return pl.pallas_call(
        _node2_body,
        grid=(N // BN,),
        in_specs=[
            pl.BlockSpec((BN, H), lambda i: (i, 0)),
            pl.BlockSpec((2, BN, H), lambda i: (0, i, 0)),
            pl.BlockSpec((H, H), lambda i: (0, 0)),
            pl.BlockSpec((1, H), lambda i: (0, 0)),
            pl.BlockSpec((H, FD), lambda i: (0, 0)),
            pl.BlockSpec((1, FD), lambda i: (0, 0)),
            pl.BlockSpec((1, FD), lambda i: (0, 0)),
            pl.BlockSpec((FD, H), lambda i: (0, 0)),
            pl.BlockSpec((H, H), lambda i: (0, 0)),
        ],
        out_specs=[
            pl.BlockSpec((BN, H), lambda i: (i, 0)),
            pl.BlockSpec((BN, H), lambda i: (i, 0)),
            pl.BlockSpec((8, 128), lambda i: (0, 0)),
        ],
        out_shape=[
            jax.ShapeDtypeStruct((N, H), jnp.float32),
            jax.ShapeDtypeStruct((N, H), jnp.float32),
            jax.ShapeDtypeStruct((8, 128), jnp.float32),
        ],
        compiler_params=pltpu.CompilerParams(dimension_semantics=("arbitrary",)),
    )(h1, aggp, l2w1p, l2b1, ow1, ob1, ow2r, ow1t, l2w1tp)


def _node3_body(gh2_ref, gp_ref, l1wtp_ref, agg0_ref, l2wp_ref, l2b_ref, l2wtp_ref, gagg0_ref):
    gproj = gp_ref[0] + gp_ref[1]
    gh1 = gh2_ref[...] + jnp.dot(gproj, l1wtp_ref[...], preferred_element_type=jnp.float32, precision=lax.Precision.HIGHEST)
    z2 = jnp.dot(agg0_ref[...], l2wp_ref[...], preferred_element_type=jnp.float32, precision=lax.Precision.HIGHEST) + l2b_ref[...]
    gagg0_ref[...] = jnp.dot(gh1 * _sig(z2), l2wtp_ref[...], preferred_element_type=jnp.float32, precision=lax.Precision.HIGHEST)


def _node3(gh2, gpp, l1w1tp, agg0, l2w0p, l2b0, l2w0tp):
    return pl.pallas_call(
        _node3_body,
        grid=(N // BN,),
        in_specs=[
            pl.BlockSpec((BN, H), lambda i: (i, 0)),
            pl.BlockSpec((2, BN, H), lambda i: (0, i, 0)),
            pl.BlockSpec((H, H), lambda i: (0, 0)),
            pl.BlockSpec((BN, H), lambda i: (i, 0)),
            pl.BlockSpec((H, H), lambda i: (0, 0)),
            pl.BlockSpec((1, H), lambda i: (0, 0)),
            pl.BlockSpec((H, H), lambda i: (0, 0)),
        ],
        out_specs=pl.BlockSpec((BN, H), lambda i: (i, 0)),
        out_shape=jax.ShapeDtypeStruct((N, H), jnp.float32),
    )(gh2, gpp, l1w1tp, agg0, l2w0p, l2b0, l2w0tp)


def _filter_bwd_body(ew_ref, gw0_ref, gw1_ref, w1_ref, b1_ref, w2p_ref, b2p_ref,
                     w1t_ref, w2tp_ref, de_ref, sig_ref):
    i = pl.program_id(0)
    ew = ew_ref[...]
    d, offs, ea, u, env = _dist_env(ew)
    gws = (gw0_ref, gw1_ref)
    g_ea = jnp.zeros((BE, NR), jnp.float32)
    g_env = jnp.zeros((BE, 1), jnp.float32)
    for t in range(2):
        a = jnp.dot(ea, w1_ref[t], preferred_element_type=jnp.float32) + b1_ref[t : t + 1, :]
        f = jnp.dot(_sp(a), w2p_ref[t], preferred_element_type=jnp.float32) + b2p_ref[t : t + 1, :]
        gw = gws[t][...]
        g_env = g_env + jnp.sum(gw * f, axis=1, keepdims=True)
        ga = jnp.dot(gw * env, w2tp_ref[t], preferred_element_type=jnp.float32) * _sig(a)
        g_ea = g_ea + jnp.dot(ga, w1t_ref[t], preferred_element_type=jnp.float32)
    dea = 2.0 * COEFF * (d - offs) * ea
    u2 = u * u
    denv = jnp.where(d <= CUT, -(30.0 * u2 * u2 - 60.0 * u2 * u + 30.0 * u2) / CUT, 0.0)
    g_d = jnp.sum(g_ea * dea, axis=1, keepdims=True) + g_env * denv
    gew = (g_d / d) * ew
    de = jnp.concatenate([gew, jnp.zeros((BE, H - 3), jnp.float32)], axis=1)
    de_ref[...] = de
    r0 = lax.broadcasted_iota(jnp.int32, (8, 128), 0) == 0
    lanes = lax.broadcasted_iota(jnp.int32, (8, 128), 1)
    total = jnp.zeros((8, 128), jnp.float32)
    for k in range(3):
        for l in range(3):
            skl = jnp.sum(ew[:, k : k + 1] * gew[:, l : l + 1])
            total = total + jnp.where(r0 & (lanes == 3 * k + l), skl, 0.0)

    @pl.when(i == 0)
    def _():
        sig_ref[...] = total

    @pl.when(i > 0)
    def _():
        sig_ref[...] = sig_ref[...] + total


def _filter_bwd(ew, gw0, gw1, w1, b1, w2p, b2p, w1t, w2tp):
    return pl.pallas_call(
        _filter_bwd_body,
        grid=(E // BE,),
        in_specs=[
            pl.BlockSpec((BE, 3), lambda i: (i, 0)),
            pl.BlockSpec((BE, H), lambda i: (i, 0)),
            pl.BlockSpec((BE, H), lambda i: (i, 0)),
            pl.BlockSpec((2, NR, FD), lambda i: (0, 0, 0)),
            pl.BlockSpec((2, FD), lambda i: (0, 0)),
            pl.BlockSpec((2, FD, H), lambda i: (0, 0, 0)),
            pl.BlockSpec((2, H), lambda i: (0, 0)),
            pl.BlockSpec((2, FD, NR), lambda i: (0, 0, 0)),
            pl.BlockSpec((2, H, FD), lambda i: (0, 0, 0)),
        ],
        out_specs=[
            pl.BlockSpec((BE, H), lambda i: (i, 0)),
            pl.BlockSpec((8, 128), lambda i: (0, 0)),
        ],
        out_shape=[
            jax.ShapeDtypeStruct((E, H), jnp.float32),
            jax.ShapeDtypeStruct((8, 128), jnp.float32),
        ],
        compiler_params=pltpu.CompilerParams(dimension_semantics=("arbitrary",)),
    )(ew, gw0, gw1, w1, b1, w2p, b2p, w1t, w2tp)


_sc_mesh = plsc.VectorSubcoreMesh(core_axis_name="c", subcore_axis_name="s")


@functools.partial(
    pl.kernel,
    out_type=jax.ShapeDtypeStruct((NC, N, H), jnp.float32),
    mesh=_sc_mesh,
    scratch_types=[
        pltpu.VMEM((CHUNK,), jnp.int32),
        pltpu.VMEM((CHUNK,), jnp.int32),
        pltpu.VMEM((CHUNK, H), jnp.float32),
        pltpu.VMEM((CHUNK, H), jnp.float32),
        pltpu.VMEM_SHARED((N, H), jnp.float32),
        pltpu.SemaphoreType.DMA,
    ],
)
def _sc_fwd(hproj, wf, ii1, jj1, zn, out, ii_c, jj_c, rows_c, wf_c, agg_sh, sem):
    cid = lax.axis_index("c")
    sid = lax.axis_index("s")
    wid = sid * NC + cid
    r0 = sid * RPS
    pltpu.sync_copy(zn.at[pl.ds(r0, RPS)], agg_sh.at[pl.ds(r0, RPS)])

    @pl.when(sid == NS - 1)
    def _():
        pltpu.sync_copy(zn.at[pl.ds(TOFF, TAIL)], agg_sh.at[pl.ds(TOFF, TAIL)])

    plsc.subcore_barrier()

    def blk(b, carry):
        cbase = wid * EPW + b * CHUNK
        pltpu.sync_copy(ii1.at[pl.ds(cbase, CHUNK)], ii_c)
        pltpu.sync_copy(jj1.at[pl.ds(cbase, CHUNK)], jj_c)
        pltpu.sync_copy(wf.at[pl.ds(cbase, CHUNK)], wf_c)
        pltpu.async_copy(hproj.at[jj_c], rows_c, sem).wait()

        def mul(e, c2):
            for c in range(FD // 16):
                s = pl.ds(c * 16, 16)
                wf_c[e, s] = rows_c[e, s] * wf_c[e, s]
            return c2

        lax.fori_loop(0, CHUNK, mul, 0)
        pltpu.sync_copy(wf_c, agg_sh.at[ii_c], add=True)
        return carry

    lax.fori_loop(0, CPW, blk, 0)
    plsc.subcore_barrier()
    pltpu.sync_copy(agg_sh.at[pl.ds(r0, RPS)], out.at[cid, pl.ds(r0, RPS)])

    @pl.when(sid == NS - 1)
    def _():
        pltpu.sync_copy(agg_sh.at[pl.ds(TOFF, TAIL)], out.at[cid, pl.ds(TOFF, TAIL)])


@functools.partial(
    pl.kernel,
    out_type=[
        jax.ShapeDtypeStruct((E, H), jnp.float32),
        jax.ShapeDtypeStruct((NC, N, H), jnp.float32),
    ],
    mesh=_sc_mesh,
    scratch_types=[
        pltpu.VMEM((CHUNK,), jnp.int32),
        pltpu.VMEM((CHUNK,), jnp.int32),
        pltpu.VMEM((CHUNK, H), jnp.float32),
        pltpu.VMEM((CHUNK, H), jnp.float32),
        pltpu.VMEM((CHUNK, H), jnp.float32),
        pltpu.VMEM_SHARED((N, H), jnp.float32),
        pltpu.SemaphoreType.DMA,
    ],
)
def _sc_bwd1(gagg, hproj, wf, ii1, jj1, zn, gw_out, gp_out,
             ii_c, jj_c, ga_c, hp_c, wf_c, gp_sh, sem):
    cid = lax.axis_index("c")
    sid = lax.axis_index("s")
    wid = sid * NC + cid
    r0 = sid * RPS
    pltpu.sync_copy(zn.at[pl.ds(r0, RPS)], gp_sh.at[pl.ds(r0, RPS)])

    @pl.when(sid == NS - 1)
    def _():
        pltpu.sync_copy(zn.at[pl.ds(TOFF, TAIL)], gp_sh.at[pl.ds(TOFF, TAIL)])

    plsc.subcore_barrier()

    def blk(b, carry):
        cbase = wid * EPW + b * CHUNK
        pltpu.sync_copy(ii1.at[pl.ds(cbase, CHUNK)], ii_c)
        pltpu.sync_copy(jj1.at[pl.ds(cbase, CHUNK)], jj_c)
        pltpu.sync_copy(wf.at[pl.ds(cbase, CHUNK)], wf_c)
        cp1 = pltpu.async_copy(gagg.at[ii_c], ga_c, sem)
        cp2 = pltpu.async_copy(hproj.at[jj_c], hp_c, sem)
        cp1.wait()
        cp2.wait()

        def mul(e, c2):
            for c in range(FD // 16):
                s = pl.ds(c * 16, 16)
                ga = ga_c[e, s]
                hp_c[e, s] = ga * hp_c[e, s]
                wf_c[e, s] = ga * wf_c[e, s]
            return c2

        lax.fori_loop(0, CHUNK, mul, 0)
        pltpu.sync_copy(hp_c, gw_out.at[pl.ds(cbase, CHUNK)])
        pltpu.sync_copy(wf_c, gp_sh.at[jj_c], add=True)
        return carry

    lax.fori_loop(0, CPW, blk, 0)
    plsc.subcore_barrier()
    pltpu.sync_copy(gp_sh.at[pl.ds(r0, RPS)], gp_out.at[cid, pl.ds(r0, RPS)])

    @pl.when(sid == NS - 1)
    def _():
        pltpu.sync_copy(gp_sh.at[pl.ds(TOFF, TAIL)], gp_out.at[cid, pl.ds(TOFF, TAIL)])


@functools.partial(
    pl.kernel,
    out_type=jax.ShapeDtypeStruct((E, H), jnp.float32),
    mesh=_sc_mesh,
    scratch_types=[
        pltpu.VMEM((CHUNK,), jnp.int32),
        pltpu.VMEM((CHUNK,), jnp.int32),
        pltpu.VMEM((CHUNK, H), jnp.float32),
        pltpu.VMEM((CHUNK, H), jnp.float32),
        pltpu.SemaphoreType.DMA,
    ],
)
def _sc_bwd0(gagg, hproj, ii1, jj1, gw_out, ii_c, jj_c, ga_c, hp_c, sem):
    cid = lax.axis_index("c")
    sid = lax.axis_index("s")
    wid = sid * NC + cid

    def blk(b, carry):
        cbase = wid * EPW + b * CHUNK
        pltpu.sync_copy(ii1.at[pl.ds(cbase, CHUNK)], ii_c)
        pltpu.sync_copy(jj1.at[pl.ds(cbase, CHUNK)], jj_c)
        cp1 = pltpu.async_copy(gagg.at[ii_c], ga_c, sem)
        cp2 = pltpu.async_copy(hproj.at[jj_c], hp_c, sem)
        cp1.wait()
        cp2.wait()

        def mul(e, c2):
            for c in range(FD // 16):
                s = pl.ds(c * 16, 16)
                hp_c[e, s] = ga_c[e, s] * hp_c[e, s]
            return c2

        lax.fori_loop(0, CHUNK, mul, 0)
        pltpu.sync_copy(hp_c, gw_out.at[pl.ds(cbase, CHUNK)])
        return carry

    lax.fori_loop(0, CPW, blk, 0)


@functools.partial(
    pl.kernel,
    out_type=jax.ShapeDtypeStruct((NC, N, H), jnp.float32),
    mesh=_sc_mesh,
    scratch_types=[
        pltpu.VMEM((CHUNK,), jnp.int32),
        pltpu.VMEM((CHUNK,), jnp.int32),
        pltpu.VMEM((CHUNK, H), jnp.float32),
        pltpu.VMEM((CHUNK, H), jnp.float32),
        pltpu.VMEM_SHARED((N, H), jnp.float32),
        pltpu.SemaphoreType.DMA,
    ],
)
def _sc_force(de, ii1, jj1, zn, out, ii_c, jj_c, de_c, nde_c, f_sh, sem):
    cid = lax.axis_index("c")
    sid = lax.axis_index("s")
    wid = sid * NC + cid
    r0 = sid * RPS
    pltpu.sync_copy(zn.at[pl.ds(r0, RPS)], f_sh.at[pl.ds(r0, RPS)])

    @pl.when(sid == NS - 1)
    def _():
        pltpu.sync_copy(zn.at[pl.ds(TOFF, TAIL)], f_sh.at[pl.ds(TOFF, TAIL)])

    def z0(e, c2):
        for c in range(H // 16):
            nde_c[e, pl.ds(c * 16, 16)] = jnp.zeros((16,), jnp.float32)
        return c2

    lax.fori_loop(0, CHUNK, z0, 0)
    plsc.subcore_barrier()

    def blk(b, carry):
        cbase = wid * EPW + b * CHUNK
        pltpu.sync_copy(ii1.at[pl.ds(cbase, CHUNK)], ii_c)
        pltpu.sync_copy(jj1.at[pl.ds(cbase, CHUNK)], jj_c)
        pltpu.sync_copy(de.at[pl.ds(cbase, CHUNK)], de_c)

        def neg(e, c2):
            nde_c[e, pl.ds(0, 16)] = -de_c[e, pl.ds(0, 16)]
            return c2

        lax.fori_loop(0, CHUNK, neg, 0)
        pltpu.sync_copy(de_c, f_sh.at[ii_c], add=True)
        pltpu.sync_copy(nde_c, f_sh.at[jj_c], add=True)
        return carry

    lax.fori_loop(0, CPW, blk, 0)
    plsc.subcore_barrier()
    pltpu.sync_copy(f_sh.at[pl.ds(r0, RPS)], out.at[cid, pl.ds(r0, RPS)])

    @pl.when(sid == NS - 1)
    def _():
        pltpu.sync_copy(f_sh.at[pl.ds(TOFF, TAIL)], out.at[cid, pl.ds(TOFF, TAIL)])


def _padc(w, cols):
    return jnp.pad(w, ((0, 0), (0, cols - w.shape[1])))


def _padr(w, rows):
    return jnp.pad(w, ((0, rows - w.shape[0]), (0, 0)))


def kernel(x, edge_index, edge_weight, emb, mlp_W1, mlp_b1, mlp_W2, mlp_b2,
           lin1_W, lin2_W, lin2_b, out_W1, out_b1, out_W2, out_b2):
    ii1 = edge_index[0].astype(jnp.int32)
    jj1 = edge_index[1].astype(jnp.int32)
    x_col = x.astype(jnp.int32).reshape(N, 1)
    emb_pad = jnp.zeros((H, H), jnp.float32).at[: emb.shape[0]].set(emb)
    zn = jnp.zeros((N, H), jnp.float32)

    w2p = jnp.pad(mlp_W2, ((0, 0), (0, 0), (0, H - FD)))
    b2p = jnp.pad(mlp_b2, ((0, 0), (0, H - FD)))
    w1t = jnp.swapaxes(mlp_W1, 1, 2)
    w2tp = jnp.pad(jnp.swapaxes(mlp_W2, 1, 2), ((0, 0), (0, H - FD), (0, 0)))
    l1w0p = _padc(lin1_W[0], H)
    l1w1p = _padc(lin1_W[1], H)
    l2w0p = _padr(lin2_W[0], H)
    l2w1p = _padr(lin2_W[1], H)
    l2w0tp = _padc(lin2_W[0].T, H)
    l2w1tp = _padc(lin2_W[1].T, H)
    l1w1tp = _padr(lin1_W[1].T, H)

    wf0, wf1 = _filter_fwd(edge_weight, mlp_W1, mlp_b1, w2p, b2p)
    h0, hp0 = _emb(x_col, emb_pad, l1w0p)
    aggp0 = _sc_fwd(hp0, wf0, ii1, jj1, zn)
    h1, hp1, agg0 = _node1(h0, aggp0, l2w0p, lin2_b[0:1], l1w1p)
    aggp1 = _sc_fwd(hp1, wf1, ii1, jj1, zn)
    gh2, gagg1, es_blk = _node2(
        h1, aggp1, l2w1p, lin2_b[1:2], out_W1, out_b1.reshape(1, FD),
        out_W2.T, out_W1.T, l2w1tp)
    gw1, gpp = _sc_bwd1(gagg1, hp1, wf1, ii1, jj1, zn)
    gagg0 = _node3(gh2, gpp, l1w1tp, agg0, l2w0p, lin2_b[0:1], l2w0tp)
    gw0 = _sc_bwd0(gagg0, hp0, ii1, jj1)
    de, sig_blk = _filter_bwd(edge_weight, gw0, gw1, mlp_W1, mlp_b1,
                              w2p, b2p, w1t, w2tp)
    fpart = _sc_force(de, ii1, jj1, zn)

    e_sum = es_blk[0, 0] + N * out_b2[0]
    forces = (fpart[0] + fpart[1])[:, :3]
    sig = sig_blk[0, :9].reshape(3, 3)
    sigma = 0.5 * (sig + sig.T)
    return (e_sum, forces, sigma)
